# Initial kernel scaffold; baseline (speedup 1.0000x reference)
import jax
import jax.numpy as jnp
from jax.experimental import pallas as pl


def kernel(x, edge_index, W1l, b1l, W1r, W2l, b2l, W2r):
    raise NotImplementedError("write your pallas kernel here")


# speedup vs baseline: 5.7328x; 5.7328x over previous
import functools

import jax
import jax.numpy as jnp
from jax import lax
from jax.experimental import pallas as pl
from jax.experimental.pallas import tpu as pltpu
from jax.experimental.pallas import tpu_sc as plsc

N = 10000
E = 320000
F_IN = 128
H = 128
C = 40
CP = 48

NC, NS = 2, 16
NW = NC * NS
EPW = E // NW
B = 80
ITERS = EPW // B
NP = 10240
RPT = NP // NS

def _sc_mesh():
    return plsc.VectorSubcoreMesh(core_axis_name="c", subcore_axis_name="s",
                                  num_cores=NC, num_subcores=NS)


@functools.cache
def _seg_sum_l1():
    @functools.partial(
        pl.kernel,
        mesh=_sc_mesh(),
        out_type=[
            jax.ShapeDtypeStruct((NC, NP, H), jnp.float32),
            jax.ShapeDtypeStruct((NC, NP, 16), jnp.float32),
        ],
        scratch_types=[
            pltpu.VMEM((B,), jnp.int32),
            pltpu.VMEM((B,), jnp.int32),
            pltpu.VMEM((B, H), jnp.float32),
            pltpu.VMEM((B, 16), jnp.float32),
            pltpu.VMEM_SHARED((NP, H), jnp.float32),
            pltpu.VMEM_SHARED((NP, 16), jnp.float32),
            pltpu.SemaphoreType.DMA,
        ],
        compiler_params=pltpu.CompilerParams(use_tc_tiling_on_sc=False),
    )
    def body_fn(xl_hbm, src_hbm, dst_hbm, z_rows_hbm, z_cnt_hbm, ones_hbm,
                part_hbm, cntp_hbm,
                src_idx, dst_idx, rows, ones_v, acc, cacc, sem):
        c = lax.axis_index("c")
        s = lax.axis_index("s")
        wid = c * NS + s
        pltpu.sync_copy(z_rows_hbm, acc.at[pl.ds(s * RPT, RPT)])
        pltpu.sync_copy(z_cnt_hbm, cacc.at[pl.ds(s * RPT, RPT)])
        pltpu.sync_copy(ones_hbm, ones_v)
        plsc.subcore_barrier()

        def body(i, carry):
            base = wid * EPW + i * B
            pltpu.sync_copy(src_hbm.at[pl.ds(base, B)], src_idx)
            pltpu.sync_copy(dst_hbm.at[pl.ds(base, B)], dst_idx)
            pltpu.async_copy(xl_hbm.at[src_idx], rows, sem).wait()
            pltpu.sync_copy(rows, acc.at[dst_idx], add=True)
            pltpu.sync_copy(ones_v, cacc.at[dst_idx], add=True)
            return carry

        lax.fori_loop(0, ITERS, body, 0)
        plsc.subcore_barrier()
        pltpu.sync_copy(acc.at[pl.ds(s * RPT, RPT)],
                        part_hbm.at[c, pl.ds(s * RPT, RPT)])
        pltpu.sync_copy(cacc.at[pl.ds(s * RPT, RPT)],
                        cntp_hbm.at[c, pl.ds(s * RPT, RPT)])

    return body_fn


@functools.cache
def _seg_sum_l2():
    @functools.partial(
        pl.kernel,
        mesh=_sc_mesh(),
        out_type=[jax.ShapeDtypeStruct((NC, NP, CP), jnp.float32)],
        scratch_types=[
            pltpu.VMEM((B,), jnp.int32),
            pltpu.VMEM((B,), jnp.int32),
            pltpu.VMEM((B, CP), jnp.float32),
            pltpu.VMEM_SHARED((NP, CP), jnp.float32),
            pltpu.SemaphoreType.DMA,
        ],
        compiler_params=pltpu.CompilerParams(use_tc_tiling_on_sc=False),
    )
    def body_fn(hl_hbm, src_hbm, dst_hbm, z_rows_hbm,
                part_hbm,
                src_idx, dst_idx, rows, acc, sem):
        c = lax.axis_index("c")
        s = lax.axis_index("s")
        wid = c * NS + s
        pltpu.sync_copy(z_rows_hbm, acc.at[pl.ds(s * RPT, RPT)])
        plsc.subcore_barrier()

        def body(i, carry):
            base = wid * EPW + i * B
            pltpu.sync_copy(src_hbm.at[pl.ds(base, B)], src_idx)
            pltpu.sync_copy(dst_hbm.at[pl.ds(base, B)], dst_idx)
            pltpu.async_copy(hl_hbm.at[src_idx], rows, sem).wait()
            pltpu.sync_copy(rows, acc.at[dst_idx], add=True)
            return carry

        lax.fori_loop(0, ITERS, body, 0)
        plsc.subcore_barrier()
        pltpu.sync_copy(acc.at[pl.ds(s * RPT, RPT)],
                        part_hbm.at[c, pl.ds(s * RPT, RPT)])

    return body_fn


BN = 1000

_DOT_T = (((1,), (1,)), ((), ()))


def _mm2_body(x_ref, wl_ref, wr_ref, xl_ref, xr_ref):
    xb = x_ref[...]
    xl_ref[...] = lax.dot_general(xb, wl_ref[...], _DOT_T,
                                  preferred_element_type=jnp.float32)
    xr_ref[...] = lax.dot_general(xb, wr_ref[...], _DOT_T,
                                  preferred_element_type=jnp.float32)


_mm2 = pl.pallas_call(
    _mm2_body,
    grid=(N // BN,),
    in_specs=[
        pl.BlockSpec((BN, F_IN), lambda i: (i, 0)),
        pl.BlockSpec((H, F_IN), lambda i: (0, 0)),
        pl.BlockSpec((H, F_IN), lambda i: (0, 0)),
    ],
    out_specs=[
        pl.BlockSpec((BN, H), lambda i: (i, 0)),
        pl.BlockSpec((BN, H), lambda i: (i, 0)),
    ],
    out_shape=[
        jax.ShapeDtypeStruct((N, H), jnp.float32),
        jax.ShapeDtypeStruct((N, H), jnp.float32),
    ],
)


def _stage_b_body(p_ref, c_ref, xr_ref, b_ref, w_ref, h_ref, hl_ref):
    tot = c_ref[0] + c_ref[1]
    den = jnp.maximum(tot[:, 0:1], 1.0)
    agg = (p_ref[0] + p_ref[1]) / den
    hb = jnp.maximum(agg + b_ref[...] + xr_ref[...], 0.0)
    h_ref[...] = hb
    hl_ref[...] = lax.dot_general(hb, w_ref[...], _DOT_T,
                                  preferred_element_type=jnp.float32)


_stage_b = pl.pallas_call(
    _stage_b_body,
    grid=(N // BN,),
    in_specs=[
        pl.BlockSpec((NC, BN, H), lambda i: (0, i, 0)),
        pl.BlockSpec((NC, BN, 16), lambda i: (0, i, 0)),
        pl.BlockSpec((BN, H), lambda i: (i, 0)),
        pl.BlockSpec((1, H), lambda i: (0, 0)),
        pl.BlockSpec((CP, H), lambda i: (0, 0)),
    ],
    out_specs=[
        pl.BlockSpec((BN, H), lambda i: (i, 0)),
        pl.BlockSpec((BN, CP), lambda i: (i, 0)),
    ],
    out_shape=[
        jax.ShapeDtypeStruct((N, H), jnp.float32),
        jax.ShapeDtypeStruct((N, CP), jnp.float32),
    ],
)


def _stage_c_body(p_ref, c_ref, h_ref, b_ref, w_ref, o_ref):
    tot = c_ref[0] + c_ref[1]
    den = jnp.maximum(tot[:, 0:1], 1.0)
    agg = (p_ref[0] + p_ref[1])[:, :C] / den
    o = agg + b_ref[...] + lax.dot_general(h_ref[...], w_ref[...], _DOT_T,
                                           preferred_element_type=jnp.float32)
    m = jnp.max(o, axis=1, keepdims=True)
    sh = o - m
    lse = jnp.log(jnp.sum(jnp.exp(sh), axis=1, keepdims=True))
    o_ref[...] = sh - lse


_stage_c = pl.pallas_call(
    _stage_c_body,
    grid=(N // BN,),
    in_specs=[
        pl.BlockSpec((NC, BN, CP), lambda i: (0, i, 0)),
        pl.BlockSpec((NC, BN, 16), lambda i: (0, i, 0)),
        pl.BlockSpec((BN, H), lambda i: (i, 0)),
        pl.BlockSpec((1, C), lambda i: (0, 0)),
        pl.BlockSpec((C, H), lambda i: (0, 0)),
    ],
    out_specs=pl.BlockSpec((BN, C), lambda i: (i, 0)),
    out_shape=jax.ShapeDtypeStruct((N, C), jnp.float32),
)


def kernel(x, edge_index, W1l, b1l, W1r, W2l, b2l, W2r):
    x = x.astype(jnp.float32)
    e_src = edge_index[0].astype(jnp.int32)
    e_dst = edge_index[1].astype(jnp.int32)

    xl, xr = _mm2(x, W1l, W1r)

    z_rows = jnp.zeros((RPT, H), jnp.float32)
    z_cnt = jnp.zeros((RPT, 16), jnp.float32)
    ones16 = jnp.ones((B, 16), jnp.float32)
    part1, cntp = _seg_sum_l1()(xl, e_src, e_dst, z_rows, z_cnt, ones16)

    W2l_pad = jnp.zeros((CP, H), jnp.float32).at[:C].set(W2l)
    h, hl = _stage_b(part1, cntp, xr, b1l.reshape(1, H), W2l_pad)

    z_rows2 = jnp.zeros((RPT, CP), jnp.float32)
    (part2,) = _seg_sum_l2()(hl, e_src, e_dst, z_rows2)

    return _stage_c(part2, cntp, h, b2l.reshape(1, C), W2r)


# speedup vs baseline: 13.3203x; 2.3235x over previous
import functools

import jax
import jax.numpy as jnp
from jax import lax
from jax.experimental import pallas as pl
from jax.experimental.pallas import tpu as pltpu
from jax.experimental.pallas import tpu_sc as plsc

N = 10000
E = 320000
F_IN = 128
H = 128
C = 40
CP = 48

NC, NS = 2, 16
NW = NC * NS
B = 80
ITERS = 125
EPW = ITERS * B
RPT = N // NS

def _sc_mesh():
    return plsc.VectorSubcoreMesh(core_axis_name="c", subcore_axis_name="s",
                                  num_cores=NC, num_subcores=NS)


@functools.cache
def _seg_sum_l1():
    @functools.partial(
        pl.kernel,
        mesh=_sc_mesh(),
        out_type=[
            jax.ShapeDtypeStruct((NC, N, H), jnp.float32),
            jax.ShapeDtypeStruct((NW, N), jnp.float32),
        ],
        scratch_types=[
            pltpu.VMEM((ITERS, B), jnp.int32),
            pltpu.VMEM((ITERS, B), jnp.int32),
            pltpu.VMEM((B, H), jnp.float32),
            pltpu.VMEM((B, H), jnp.float32),
            pltpu.VMEM((N,), jnp.float32),
            pltpu.VMEM_SHARED((N, H), jnp.float32),
            pltpu.SemaphoreType.DMA,
            pltpu.SemaphoreType.DMA,
        ],
        compiler_params=pltpu.CompilerParams(use_tc_tiling_on_sc=False,
                                             needs_layout_passes=False),
    )
    def body_fn(xl_hbm, src_hbm, dst_hbm, z_rows_hbm, z_hist_hbm,
                part_hbm, cntp_hbm,
                src2, dst2, rows0, rows1, hist, acc, sem0, sem1):
        c = lax.axis_index("c")
        s = lax.axis_index("s")
        wid = c * NS + s
        ones16 = jnp.ones((16,), jnp.float32)
        pltpu.sync_copy(src_hbm.at[pl.ds(wid * ITERS, ITERS)], src2)
        pltpu.sync_copy(dst_hbm.at[pl.ds(wid * ITERS, ITERS)], dst2)
        pltpu.sync_copy(z_rows_hbm, acc.at[pl.ds(s * RPT, RPT)])
        pltpu.sync_copy(z_hist_hbm, hist)
        plsc.subcore_barrier()

        def count(ii):
            for k in range(B // 16):
                idx = dst2[ii, pl.ds(k * 16, 16)]
                plsc.addupdate_scatter(hist, [idx], ones16)

        pltpu.async_copy(xl_hbm.at[src2.at[0]], rows0, sem0)

        def body(j, carry):
            i0 = 2 * j
            i1 = i0 + 1
            pltpu.async_copy(xl_hbm.at[src2.at[i1]], rows1, sem1)
            pltpu.make_async_copy(xl_hbm.at[src2.at[i0]], rows0, sem0).wait()
            pltpu.sync_copy(rows0, acc.at[dst2.at[i0]], add=True)
            count(i0)
            pltpu.async_copy(xl_hbm.at[src2.at[i0 + 2]], rows0, sem0)
            pltpu.make_async_copy(xl_hbm.at[src2.at[i1]], rows1, sem1).wait()
            pltpu.sync_copy(rows1, acc.at[dst2.at[i1]], add=True)
            count(i1)
            return carry

        lax.fori_loop(0, ITERS // 2, body, 0)
        last = ITERS - 1
        pltpu.make_async_copy(xl_hbm.at[src2.at[last]], rows0, sem0).wait()
        pltpu.sync_copy(rows0, acc.at[dst2.at[last]], add=True)
        count(last)
        plsc.subcore_barrier()
        pltpu.sync_copy(acc.at[pl.ds(s * RPT, RPT)],
                        part_hbm.at[c, pl.ds(s * RPT, RPT)])
        pltpu.sync_copy(hist, cntp_hbm.at[wid])

    return body_fn


@functools.cache
def _seg_sum_l2():
    @functools.partial(
        pl.kernel,
        mesh=_sc_mesh(),
        out_type=[jax.ShapeDtypeStruct((NC, N, CP), jnp.float32)],
        scratch_types=[
            pltpu.VMEM((ITERS, B), jnp.int32),
            pltpu.VMEM((ITERS, B), jnp.int32),
            pltpu.VMEM((B, CP), jnp.float32),
            pltpu.VMEM((B, CP), jnp.float32),
            pltpu.VMEM_SHARED((N, CP), jnp.float32),
            pltpu.SemaphoreType.DMA,
            pltpu.SemaphoreType.DMA,
        ],
        compiler_params=pltpu.CompilerParams(use_tc_tiling_on_sc=False,
                                             needs_layout_passes=False),
    )
    def body_fn(hl_hbm, src_hbm, dst_hbm, z_rows_hbm,
                part_hbm,
                src2, dst2, rows0, rows1, acc, sem0, sem1):
        c = lax.axis_index("c")
        s = lax.axis_index("s")
        wid = c * NS + s
        pltpu.sync_copy(src_hbm.at[pl.ds(wid * ITERS, ITERS)], src2)
        pltpu.sync_copy(dst_hbm.at[pl.ds(wid * ITERS, ITERS)], dst2)
        pltpu.sync_copy(z_rows_hbm, acc.at[pl.ds(s * RPT, RPT)])
        plsc.subcore_barrier()

        pltpu.async_copy(hl_hbm.at[src2.at[0]], rows0, sem0)

        def body(j, carry):
            i0 = 2 * j
            i1 = i0 + 1
            pltpu.async_copy(hl_hbm.at[src2.at[i1]], rows1, sem1)
            pltpu.make_async_copy(hl_hbm.at[src2.at[i0]], rows0, sem0).wait()
            pltpu.sync_copy(rows0, acc.at[dst2.at[i0]], add=True)
            pltpu.async_copy(hl_hbm.at[src2.at[i0 + 2]], rows0, sem0)
            pltpu.make_async_copy(hl_hbm.at[src2.at[i1]], rows1, sem1).wait()
            pltpu.sync_copy(rows1, acc.at[dst2.at[i1]], add=True)
            return carry

        lax.fori_loop(0, ITERS // 2, body, 0)
        last = ITERS - 1
        pltpu.make_async_copy(hl_hbm.at[src2.at[last]], rows0, sem0).wait()
        pltpu.sync_copy(rows0, acc.at[dst2.at[last]], add=True)
        plsc.subcore_barrier()
        pltpu.sync_copy(acc.at[pl.ds(s * RPT, RPT)],
                        part_hbm.at[c, pl.ds(s * RPT, RPT)])

    return body_fn


BN = 1280

_DOT_T = (((1,), (1,)), ((), ()))


def _mm2_body(x_ref, wl_ref, wr_ref, xl_ref, xr_ref):
    xb = x_ref[...]
    xl_ref[...] = lax.dot_general(xb, wl_ref[...], _DOT_T,
                                  preferred_element_type=jnp.float32)
    xr_ref[...] = lax.dot_general(xb, wr_ref[...], _DOT_T,
                                  preferred_element_type=jnp.float32)


_mm2 = pl.pallas_call(
    _mm2_body,
    grid=(pl.cdiv(N, BN),),
    in_specs=[
        pl.BlockSpec((BN, F_IN), lambda i: (i, 0)),
        pl.BlockSpec((H, F_IN), lambda i: (0, 0)),
        pl.BlockSpec((H, F_IN), lambda i: (0, 0)),
    ],
    out_specs=[
        pl.BlockSpec((BN, H), lambda i: (i, 0)),
        pl.BlockSpec((BN, H), lambda i: (i, 0)),
    ],
    out_shape=[
        jax.ShapeDtypeStruct((N, H), jnp.float32),
        jax.ShapeDtypeStruct((N, H), jnp.float32),
    ],
)


def _stage_b_body(p_ref, c_ref, xr_ref, b_ref, w_ref, h_ref, hl_ref):
    tot = jnp.sum(c_ref[...], axis=0)
    den = jnp.maximum(tot, 1.0)[:, None]
    agg = (p_ref[0] + p_ref[1]) / den
    hb = jnp.maximum(agg + b_ref[...] + xr_ref[...], 0.0)
    h_ref[...] = hb
    hl_ref[...] = lax.dot_general(hb, w_ref[...], _DOT_T,
                                  preferred_element_type=jnp.float32)


_stage_b = pl.pallas_call(
    _stage_b_body,
    grid=(pl.cdiv(N, BN),),
    in_specs=[
        pl.BlockSpec((NC, BN, H), lambda i: (0, i, 0)),
        pl.BlockSpec((NW, BN), lambda i: (0, i)),
        pl.BlockSpec((BN, H), lambda i: (i, 0)),
        pl.BlockSpec((1, H), lambda i: (0, 0)),
        pl.BlockSpec((CP, H), lambda i: (0, 0)),
    ],
    out_specs=[
        pl.BlockSpec((BN, H), lambda i: (i, 0)),
        pl.BlockSpec((BN, CP), lambda i: (i, 0)),
    ],
    out_shape=[
        jax.ShapeDtypeStruct((N, H), jnp.float32),
        jax.ShapeDtypeStruct((N, CP), jnp.float32),
    ],
)


def _stage_c_body(p_ref, c_ref, h_ref, b_ref, w_ref, o_ref):
    tot = jnp.sum(c_ref[...], axis=0)
    den = jnp.maximum(tot, 1.0)[:, None]
    agg = (p_ref[0] + p_ref[1])[:, :C] / den
    o = agg + b_ref[...] + lax.dot_general(h_ref[...], w_ref[...], _DOT_T,
                                           preferred_element_type=jnp.float32)
    m = jnp.max(o, axis=1, keepdims=True)
    sh = o - m
    lse = jnp.log(jnp.sum(jnp.exp(sh), axis=1, keepdims=True))
    o_ref[...] = sh - lse


_stage_c = pl.pallas_call(
    _stage_c_body,
    grid=(pl.cdiv(N, BN),),
    in_specs=[
        pl.BlockSpec((NC, BN, CP), lambda i: (0, i, 0)),
        pl.BlockSpec((NW, BN), lambda i: (0, i)),
        pl.BlockSpec((BN, H), lambda i: (i, 0)),
        pl.BlockSpec((1, C), lambda i: (0, 0)),
        pl.BlockSpec((C, H), lambda i: (0, 0)),
    ],
    out_specs=pl.BlockSpec((BN, C), lambda i: (i, 0)),
    out_shape=jax.ShapeDtypeStruct((N, C), jnp.float32),
)


def kernel(x, edge_index, W1l, b1l, W1r, W2l, b2l, W2r):
    x = x.astype(jnp.float32)
    e_src = edge_index[0].astype(jnp.int32).reshape(NW * ITERS, B)
    e_dst = edge_index[1].astype(jnp.int32).reshape(NW * ITERS, B)

    xl, xr = _mm2(x, W1l, W1r)

    z_rows = jnp.zeros((RPT, H), jnp.float32)
    z_hist = jnp.zeros((N,), jnp.float32)
    part1, cntp = _seg_sum_l1()(xl, e_src, e_dst, z_rows, z_hist)

    W2l_pad = jnp.zeros((CP, H), jnp.float32).at[:C].set(W2l)
    h, hl = _stage_b(part1, cntp, xr, b1l.reshape(1, H), W2l_pad)

    z_rows2 = jnp.zeros((RPT, CP), jnp.float32)
    (part2,) = _seg_sum_l2()(hl, e_src, e_dst, z_rows2)

    return _stage_c(part2, cntp, h, b2l.reshape(1, C), W2r)
